# baseline (speedup 1.0000x reference)
import functools
import math

import jax
import jax.numpy as jnp
from jax import lax
from jax.experimental import pallas as pl
from jax.experimental.pallas import tpu as pltpu
from jax.experimental.pallas import tpu_sc as plsc

_SIZE = 32000
_PAD = 0
_SMOOTH = 0.1
_CONF = 1.0 - _SMOOTH
_FILL = _SMOOTH / (_SIZE - 2)
_C0 = _CONF * math.log(_CONF) + _SMOOTH * math.log(_FILL)

_ROWS = 128

_NC, _NS, _L = 2, 16, 16
_NW = _NC * _NS
_N = 2048
_BPW = _N // _NW


def _tc_body(t_ref, x_ref, out_ref):
    j = pl.program_id(0)
    x = x_ref[...]
    r, size = x.shape
    t = t_ref[:, 0]

    p = jnp.zeros((r, 128), jnp.float32)
    for k in range(size // 128):
        p = p + x[:, k * 128:(k + 1) * 128]

    ones = jnp.ones((128, 1), jnp.float32)
    rs = jax.lax.dot(p, ones, preferred_element_type=jnp.float32)[:, 0]
    validf = (t != _PAD).astype(jnp.float32)
    partial = jnp.sum(validf * rs)

    @pl.when(j == 0)
    def _():
        out_ref[...] = partial.reshape(1, 1)

    @pl.when(j > 0)
    def _():
        out_ref[...] += partial.reshape(1, 1)


_sc_mesh = plsc.VectorSubcoreMesh(
    core_axis_name="c", subcore_axis_name="s",
    num_cores=_NC, num_subcores=_NS)


@functools.partial(
    pl.kernel,
    out_type=(jax.ShapeDtypeStruct((_NW, _L), jnp.float32),
              jax.ShapeDtypeStruct((_NW, _L), jnp.float32),
              jax.ShapeDtypeStruct((_NW, _L), jnp.float32)),
    mesh=_sc_mesh,
    scratch_types=[
        pltpu.VMEM((_BPW,), jnp.int32),
        pltpu.VMEM((_BPW,), jnp.int32),
        pltpu.VMEM((_BPW,), jnp.int32),
        pltpu.VMEM((_BPW,), jnp.float32),
        pltpu.VMEM((_BPW,), jnp.float32),
        pltpu.VMEM((_L,), jnp.float32),
        pltpu.VMEM((_L,), jnp.float32),
        pltpu.VMEM((_L,), jnp.float32),
        pltpu.SemaphoreType.DMA,
    ],
)
def _sc_gather(xf_hbm, tgt_hbm, out_t, out_0, out_n,
               tgt_v, idx_v, idx0_v, gt_v, g0_v, at_v, a0_v, an_v, sem):
    wid = lax.axis_index("s") * _NC + lax.axis_index("c")
    base = wid * _BPW
    pltpu.sync_copy(tgt_hbm.at[pl.ds(base, _BPW)], tgt_v)
    for g in range(_BPW // _L):
        tv = tgt_v[pl.ds(g * _L, _L)]
        rowv = base + g * _L + lax.iota(jnp.int32, _L)
        idx_v[pl.ds(g * _L, _L)] = rowv * _SIZE + tv
        idx0_v[pl.ds(g * _L, _L)] = rowv * _SIZE
    pltpu.async_copy(xf_hbm.at[idx_v], gt_v, sem).wait()
    pltpu.async_copy(xf_hbm.at[idx0_v], g0_v, sem).wait()
    acc_t = jnp.zeros((_L,), jnp.float32)
    acc_0 = jnp.zeros((_L,), jnp.float32)
    acc_n = jnp.zeros((_L,), jnp.float32)
    for g in range(_BPW // _L):
        sl = pl.ds(g * _L, _L)
        valid = tgt_v[sl] != _PAD
        acc_t += jnp.where(valid, gt_v[sl], 0.0)
        acc_0 += jnp.where(valid, g0_v[sl], 0.0)
        acc_n += jnp.where(valid, 1.0, 0.0)
    at_v[...] = acc_t
    a0_v[...] = acc_0
    an_v[...] = acc_n
    pltpu.sync_copy(at_v, out_t.at[wid])
    pltpu.sync_copy(a0_v, out_0.at[wid])
    pltpu.sync_copy(an_v, out_n.at[wid])


@jax.jit
def kernel(x, target):
    n, size = x.shape
    t2 = target.reshape(n, 1)
    grid = n // _ROWS
    s_dense = pl.pallas_call(
        _tc_body,
        grid=(grid,),
        in_specs=[
            pl.BlockSpec((_ROWS, 1), lambda j: (j, 0)),
            pl.BlockSpec((_ROWS, size), lambda j: (j, 0)),
        ],
        out_specs=pl.BlockSpec((1, 1), lambda j: (0, 0)),
        out_shape=jax.ShapeDtypeStruct((1, 1), jnp.float32),
    )(t2, x)[0, 0]

    parts_t, parts_0, parts_n = _sc_gather(x.reshape(-1), target)
    s_t = jnp.sum(parts_t)
    s_0 = jnp.sum(parts_0)
    nv = jnp.sum(parts_n)
    return nv * _C0 - (_CONF - _FILL) * s_t - _FILL * (s_dense - s_0)
